# initial kernel scaffold (unmeasured)
import functools

import jax
import jax.numpy as jnp
from jax import lax
from jax.experimental import pallas as pl
from jax.experimental.pallas import tpu as pltpu

N_DEV = 8
STAGE_XOR = (1, 3, 4)
SCALE = 0.08838834764831843


def kernel(x, Wq, Wo, K_ext, V_ext):
    B, Sq, D = x.shape
    _, Ckv, H, Dh = K_ext.shape
    BH = B * H

    def body(x_ref, wq_ref, wo_ref, k_ref, v_ref, out_ref,
             o_state, ml_state, comm_o, comm_ml, attn_ref,
             so, ro, sml, rml):
        my = lax.axis_index("i")

        barrier = pltpu.get_barrier_semaphore()
        for xr in STAGE_XOR:
            pl.semaphore_signal(
                barrier, inc=1,
                device_id=(my ^ xr,), device_id_type=pl.DeviceIdType.MESH,
            )

        wq = wq_ref[:].astype(jnp.bfloat16)
        for b in range(B):
            xb = x_ref[b].astype(jnp.bfloat16)
            q = lax.dot_general(
                xb, wq, (((1,), (0,)), ((), ())),
                preferred_element_type=jnp.float32,
            )
            q = (q * SCALE).astype(jnp.bfloat16)
            for h in range(H):
                idx = b * H + h
                qh = q[:, h * Dh:(h + 1) * Dh]
                kh = k_ref[b, :, h, :].astype(jnp.bfloat16)
                vh = v_ref[b, :, h, :].astype(jnp.bfloat16)
                s = lax.dot_general(
                    qh, kh, (((1,), (1,)), ((), ())),
                    preferred_element_type=jnp.float32,
                )
                m = jnp.max(s, axis=1, keepdims=True)
                p = jnp.exp(s - m)
                l = jnp.sum(p, axis=1, keepdims=True)
                o = lax.dot_general(
                    p.astype(jnp.bfloat16), vh, (((1,), (0,)), ((), ())),
                    preferred_element_type=jnp.float32,
                )
                o_state[idx] = o
                ml_state[:, idx:idx + 1] = m
                ml_state[:, BH + idx:BH + idx + 1] = l

        pl.semaphore_wait(barrier, len(STAGE_XOR))

        for s_i, xr in enumerate(STAGE_XOR):
            partner = my ^ xr
            rdma_o = pltpu.make_async_remote_copy(
                src_ref=o_state,
                dst_ref=comm_o.at[s_i],
                send_sem=so.at[s_i],
                recv_sem=ro.at[s_i],
                device_id=(partner,),
                device_id_type=pl.DeviceIdType.MESH,
            )
            rdma_ml = pltpu.make_async_remote_copy(
                src_ref=ml_state,
                dst_ref=comm_ml.at[s_i],
                send_sem=sml.at[s_i],
                recv_sem=rml.at[s_i],
                device_id=(partner,),
                device_id_type=pl.DeviceIdType.MESH,
            )
            rdma_o.start()
            rdma_ml.start()
            rdma_o.wait()
            rdma_ml.wait()

            m_mine = ml_state[:, 0:BH]
            l_mine = ml_state[:, BH:2 * BH]
            m_oth = comm_ml[s_i, :, 0:BH]
            l_oth = comm_ml[s_i, :, BH:2 * BH]
            m_new = jnp.maximum(m_mine, m_oth)
            a_mine = jnp.exp(m_mine - m_new)
            a_oth = jnp.exp(m_oth - m_new)
            ml_state[:, 0:BH] = m_new
            ml_state[:, BH:2 * BH] = l_mine * a_mine + l_oth * a_oth
            for idx in range(BH):
                am = a_mine[:, idx:idx + 1]
                ao = a_oth[:, idx:idx + 1]
                o_state[idx] = o_state[idx] * am + comm_o[s_i, idx] * ao

        wo = wo_ref[:].astype(jnp.bfloat16)
        l_all = ml_state[:, BH:2 * BH]
        for b in range(B):
            for h in range(H):
                idx = b * H + h
                inv_l = 1.0 / l_all[:, idx:idx + 1]
                attn_ref[:, h * Dh:(h + 1) * Dh] = (
                    o_state[idx] * inv_l
                ).astype(jnp.bfloat16)
            out_ref[b] = lax.dot_general(
                attn_ref[:], wo, (((1,), (0,)), ((), ())),
                preferred_element_type=jnp.float32,
            )

        @functools.partial(
            pl.run_scoped, second_barrier=pltpu.SemaphoreType.REGULAR
        )
        def _(second_barrier):
            for xr in STAGE_XOR:
                pl.semaphore_signal(
                    second_barrier, inc=1,
                    device_id=(my ^ xr,), device_id_type=pl.DeviceIdType.MESH,
                )
            pl.semaphore_wait(second_barrier, len(STAGE_XOR))

    return pl.pallas_call(
        body,
        out_shape=jax.ShapeDtypeStruct((B, Sq, D), jnp.float32),
        in_specs=[pl.BlockSpec(memory_space=pltpu.VMEM)] * 5,
        out_specs=pl.BlockSpec(memory_space=pltpu.VMEM),
        scratch_shapes=[
            pltpu.VMEM((BH, Sq, Dh), jnp.float32),
            pltpu.VMEM((Sq, 2 * BH), jnp.float32),
            pltpu.VMEM((3, BH, Sq, Dh), jnp.float32),
            pltpu.VMEM((3, Sq, 2 * BH), jnp.float32),
            pltpu.VMEM((Sq, D), jnp.bfloat16),
            pltpu.SemaphoreType.DMA((3,)),
            pltpu.SemaphoreType.DMA((3,)),
            pltpu.SemaphoreType.DMA((3,)),
            pltpu.SemaphoreType.DMA((3,)),
        ],
        compiler_params=pltpu.CompilerParams(collective_id=0),
    )(x, Wq, Wo, K_ext, V_ext)


# baseline (device time: 163432 ns/iter reference)
import functools

import jax
import jax.numpy as jnp
from jax import lax
from jax.experimental import pallas as pl
from jax.experimental.pallas import tpu as pltpu

N_DEV = 8
STAGE_XOR = (1, 3, 4)
SCALE = 0.08838834764831843


def kernel(x, Wq, Wo, K_ext, V_ext):
    B, Sq, D = x.shape
    _, Ckv, H, Dh = K_ext.shape
    BH = B * H

    def body(x_ref, wq_ref, wo_ref, k_ref, v_ref, out_ref,
             o_state, o_send, ml_state, comm_o, comm_ml, attn_ref,
             so, ro, sml, rml):
        my = lax.axis_index("i")

        barrier = pltpu.get_barrier_semaphore()
        for xr in STAGE_XOR:
            pl.semaphore_signal(
                barrier, inc=1,
                device_id=(my ^ xr,), device_id_type=pl.DeviceIdType.MESH,
            )

        def local_b(b, carry):
            xb = x_ref[b].astype(jnp.bfloat16)
            wq = wq_ref[:].astype(jnp.bfloat16)
            q = lax.dot_general(
                xb, wq, (((1,), (0,)), ((), ())),
                preferred_element_type=jnp.float32,
            )
            q = (q * SCALE).astype(jnp.bfloat16)
            for h in range(H):
                qh = q[:, h * Dh:(h + 1) * Dh]
                kh = k_ref[b, h]
                vh = v_ref[b, h]
                s = lax.dot_general(
                    qh, kh, (((1,), (1,)), ((), ())),
                    preferred_element_type=jnp.float32,
                )
                m = jnp.max(s, axis=1, keepdims=True)
                p = jnp.exp(s - m)
                l = jnp.sum(p, axis=1, keepdims=True)
                o = lax.dot_general(
                    p.astype(jnp.bfloat16), vh, (((1,), (0,)), ((), ())),
                    preferred_element_type=jnp.float32,
                )
                o_state[b * H + h] = o
                o_send[b * H + h] = o.astype(jnp.bfloat16)
                ml_state[b, :, h:h + 1] = m
                ml_state[b, :, H + h:H + h + 1] = l
            return carry

        lax.fori_loop(0, B, local_b, 0)

        pl.semaphore_wait(barrier, len(STAGE_XOR))

        for s_i, xr in enumerate(STAGE_XOR):
            partner = my ^ xr
            rdma_o = pltpu.make_async_remote_copy(
                src_ref=o_send,
                dst_ref=comm_o.at[s_i],
                send_sem=so.at[s_i],
                recv_sem=ro.at[s_i],
                device_id=(partner,),
                device_id_type=pl.DeviceIdType.MESH,
            )
            rdma_ml = pltpu.make_async_remote_copy(
                src_ref=ml_state,
                dst_ref=comm_ml.at[s_i],
                send_sem=sml.at[s_i],
                recv_sem=rml.at[s_i],
                device_id=(partner,),
                device_id_type=pl.DeviceIdType.MESH,
            )
            rdma_o.start()
            rdma_ml.start()
            rdma_o.wait()
            rdma_ml.wait()

            last = s_i == len(STAGE_XOR) - 1

            def merge_b(b, carry, s_i=s_i, last=last):
                m_mine = ml_state[b, :, 0:H]
                l_mine = ml_state[b, :, H:2 * H]
                m_oth = comm_ml[s_i, b, :, 0:H]
                l_oth = comm_ml[s_i, b, :, H:2 * H]
                m_new = jnp.maximum(m_mine, m_oth)
                a_mine = jnp.exp(m_mine - m_new)
                a_oth = jnp.exp(m_oth - m_new)
                ml_state[b, :, 0:H] = m_new
                ml_state[b, :, H:2 * H] = l_mine * a_mine + l_oth * a_oth
                for h in range(H):
                    idx = b * H + h
                    merged = (
                        o_state[idx] * a_mine[:, h:h + 1]
                        + comm_o[s_i, idx].astype(jnp.float32)
                        * a_oth[:, h:h + 1]
                    )
                    o_state[idx] = merged
                    if not last:
                        o_send[idx] = merged.astype(jnp.bfloat16)
                return carry

            lax.fori_loop(0, B, merge_b, 0)

        def final_b(b, carry):
            for h in range(H):
                inv_l = 1.0 / ml_state[b, :, H + h:H + h + 1]
                attn_ref[:, h * Dh:(h + 1) * Dh] = (
                    o_state[b * H + h] * inv_l
                ).astype(jnp.bfloat16)
            wo = wo_ref[:].astype(jnp.bfloat16)
            out_ref[b] = lax.dot_general(
                attn_ref[:], wo, (((1,), (0,)), ((), ())),
                preferred_element_type=jnp.float32,
            )
            return carry

        lax.fori_loop(0, B, final_b, 0)

        @functools.partial(
            pl.run_scoped, second_barrier=pltpu.SemaphoreType.REGULAR
        )
        def _(second_barrier):
            for xr in STAGE_XOR:
                pl.semaphore_signal(
                    second_barrier, inc=1,
                    device_id=(my ^ xr,), device_id_type=pl.DeviceIdType.MESH,
                )
            pl.semaphore_wait(second_barrier, len(STAGE_XOR))

    k_t = K_ext.astype(jnp.bfloat16).transpose(0, 2, 1, 3)
    v_t = V_ext.astype(jnp.bfloat16).transpose(0, 2, 1, 3)

    return pl.pallas_call(
        body,
        out_shape=jax.ShapeDtypeStruct((B, Sq, D), jnp.float32),
        in_specs=[pl.BlockSpec(memory_space=pltpu.VMEM)] * 5,
        out_specs=pl.BlockSpec(memory_space=pltpu.VMEM),
        scratch_shapes=[
            pltpu.VMEM((BH, Sq, Dh), jnp.float32),
            pltpu.VMEM((BH, Sq, Dh), jnp.bfloat16),
            pltpu.VMEM((B, Sq, 2 * H), jnp.float32),
            pltpu.VMEM((3, BH, Sq, Dh), jnp.bfloat16),
            pltpu.VMEM((3, B, Sq, 2 * H), jnp.float32),
            pltpu.VMEM((Sq, D), jnp.bfloat16),
            pltpu.SemaphoreType.DMA((3,)),
            pltpu.SemaphoreType.DMA((3,)),
            pltpu.SemaphoreType.DMA((3,)),
            pltpu.SemaphoreType.DMA((3,)),
        ],
        compiler_params=pltpu.CompilerParams(
            collective_id=0, vmem_limit_bytes=63 * 1024 * 1024
        ),
    )(x, Wq, Wo, k_t, v_t)


# device time: 109827 ns/iter; 1.4881x vs baseline; 1.4881x over previous
import functools

import jax
import jax.numpy as jnp
from jax import lax
from jax.experimental import pallas as pl
from jax.experimental.pallas import tpu as pltpu

N_DEV = 8
STAGE_XOR = (1, 3, 4)
SCALE = 0.08838834764831843


def kernel(x, Wq, Wo, K_ext, V_ext):
    B, Sq, D = x.shape
    _, Ckv, H, Dh = K_ext.shape
    BH = B * H
    N_STAGE = len(STAGE_XOR)

    def body(x_ref, wq_ref, wo_ref, k_ref, v_ref, out_ref,
             o_state, o_send, ml_state, ml_send, comm_o, comm_ml,
             q_scratch, attn_ref, so, ro, sml, rml):
        my = lax.axis_index("i")

        barrier = pltpu.get_barrier_semaphore()
        for xr in STAGE_XOR:
            pl.semaphore_signal(
                barrier, inc=1,
                device_id=(my ^ xr,), device_id_type=pl.DeviceIdType.MESH,
            )
        pl.semaphore_wait(barrier, N_STAGE)

        def rdma_pair(s_i, c, partner):
            r_o = pltpu.make_async_remote_copy(
                src_ref=o_send.at[pl.ds(c * H, H)],
                dst_ref=comm_o.at[s_i, pl.ds(c * H, H)],
                send_sem=so.at[s_i, c],
                recv_sem=ro.at[s_i, c],
                device_id=(partner,),
                device_id_type=pl.DeviceIdType.MESH,
            )
            r_ml = pltpu.make_async_remote_copy(
                src_ref=ml_send.at[c],
                dst_ref=comm_ml.at[s_i, c],
                send_sem=sml.at[s_i, c],
                recv_sem=rml.at[s_i, c],
                device_id=(partner,),
                device_id_type=pl.DeviceIdType.MESH,
            )
            return r_o, r_ml

        x_all = jnp.reshape(x_ref[:], (B * Sq, D)).astype(jnp.bfloat16)
        wq = wq_ref[:].astype(jnp.bfloat16)
        q_all = lax.dot_general(
            x_all, wq, (((1,), (0,)), ((), ())),
            preferred_element_type=jnp.float32,
        )
        q_scratch[:] = (q_all * SCALE).astype(jnp.bfloat16)

        def local_b(b, carry):
            q = q_scratch[pl.ds(b * Sq, Sq), :]
            for h in range(H):
                qh = q[:, h * Dh:(h + 1) * Dh]
                kh = k_ref[b, h]
                vh = v_ref[b, h]
                s = lax.dot_general(
                    qh, kh, (((1,), (1,)), ((), ())),
                    preferred_element_type=jnp.float32,
                )
                m = jnp.max(s, axis=1, keepdims=True)
                p = jnp.exp(s - m)
                l = jnp.sum(p, axis=1, keepdims=True)
                o = lax.dot_general(
                    p.astype(jnp.bfloat16), vh, (((1,), (0,)), ((), ())),
                    preferred_element_type=jnp.float32,
                )
                o_state[b * H + h] = o
                o_send[b * H + h] = o.astype(jnp.bfloat16)
                ml_state[b, :, h:h + 1] = m
                ml_state[b, :, H + h:H + h + 1] = l
                ml_send[b, :, h:h + 1] = m.astype(jnp.bfloat16)
                ml_send[b, :, H + h:H + h + 1] = l.astype(jnp.bfloat16)
            return carry

        lax.fori_loop(0, B // 2, local_b, 0)
        for c in range(B // 2):
            r_o, r_ml = rdma_pair(0, c, my ^ STAGE_XOR[0])
            r_o.start()
            r_ml.start()
        lax.fori_loop(B // 2, B, local_b, 0)
        for c in range(B // 2, B):
            r_o, r_ml = rdma_pair(0, c, my ^ STAGE_XOR[0])
            r_o.start()
            r_ml.start()

        for s_i, xr in enumerate(STAGE_XOR):
            last = s_i == N_STAGE - 1
            for c in range(B):
                r_o, r_ml = rdma_pair(s_i, c, my ^ xr)
                r_o.wait()
                r_ml.wait()

                def merge_b(b, carry, s_i=s_i, last=last):
                    m_mine = ml_state[b, :, 0:H]
                    l_mine = ml_state[b, :, H:2 * H]
                    m_oth = comm_ml[s_i, b, :, 0:H].astype(jnp.float32)
                    l_oth = comm_ml[s_i, b, :, H:2 * H].astype(jnp.float32)
                    m_new = jnp.maximum(m_mine, m_oth)
                    a_mine = jnp.exp(m_mine - m_new)
                    a_oth = jnp.exp(m_oth - m_new)
                    l_new = l_mine * a_mine + l_oth * a_oth
                    ml_state[b, :, 0:H] = m_new
                    ml_state[b, :, H:2 * H] = l_new
                    if not last:
                        ml_send[b, :, 0:H] = m_new.astype(jnp.bfloat16)
                        ml_send[b, :, H:2 * H] = l_new.astype(jnp.bfloat16)
                    for h in range(H):
                        idx = b * H + h
                        merged = (
                            o_state[idx] * a_mine[:, h:h + 1]
                            + comm_o[s_i, idx].astype(jnp.float32)
                            * a_oth[:, h:h + 1]
                        )
                        o_state[idx] = merged
                        if not last:
                            o_send[idx] = merged.astype(jnp.bfloat16)
                    return carry

                lax.fori_loop(c, c + 1, merge_b, 0)
                if not last:
                    n_o, n_ml = rdma_pair(s_i + 1, c, my ^ STAGE_XOR[s_i + 1])
                    n_o.start()
                    n_ml.start()

        def final_b(b, carry):
            for h in range(H):
                inv_l = 1.0 / ml_state[b, :, H + h:H + h + 1]
                attn_ref[pl.ds(b * Sq, Sq), h * Dh:(h + 1) * Dh] = (
                    o_state[b * H + h] * inv_l
                ).astype(jnp.bfloat16)
            return carry

        lax.fori_loop(0, B, final_b, 0)
        wo = wo_ref[:].astype(jnp.bfloat16)
        out = lax.dot_general(
            attn_ref[:], wo, (((1,), (0,)), ((), ())),
            preferred_element_type=jnp.float32,
        )
        out_ref[:] = jnp.reshape(out, (B, Sq, D))

        @functools.partial(
            pl.run_scoped, second_barrier=pltpu.SemaphoreType.REGULAR
        )
        def _(second_barrier):
            for xr in STAGE_XOR:
                pl.semaphore_signal(
                    second_barrier, inc=1,
                    device_id=(my ^ xr,), device_id_type=pl.DeviceIdType.MESH,
                )
            pl.semaphore_wait(second_barrier, N_STAGE)

    k_t = K_ext.astype(jnp.bfloat16).transpose(0, 2, 1, 3)
    v_t = V_ext.astype(jnp.bfloat16).transpose(0, 2, 1, 3)

    return pl.pallas_call(
        body,
        out_shape=jax.ShapeDtypeStruct((B, Sq, D), jnp.float32),
        in_specs=[pl.BlockSpec(memory_space=pltpu.VMEM)] * 5,
        out_specs=pl.BlockSpec(memory_space=pltpu.VMEM),
        scratch_shapes=[
            pltpu.VMEM((BH, Sq, Dh), jnp.float32),
            pltpu.VMEM((BH, Sq, Dh), jnp.bfloat16),
            pltpu.VMEM((B, Sq, 2 * H), jnp.float32),
            pltpu.VMEM((B, Sq, 2 * H), jnp.bfloat16),
            pltpu.VMEM((N_STAGE, BH, Sq, Dh), jnp.bfloat16),
            pltpu.VMEM((N_STAGE, B, Sq, 2 * H), jnp.bfloat16),
            pltpu.VMEM((B * Sq, D), jnp.bfloat16),
            pltpu.VMEM((B * Sq, D), jnp.bfloat16),
            pltpu.SemaphoreType.DMA((N_STAGE, B)),
            pltpu.SemaphoreType.DMA((N_STAGE, B)),
            pltpu.SemaphoreType.DMA((N_STAGE, B)),
            pltpu.SemaphoreType.DMA((N_STAGE, B)),
        ],
        compiler_params=pltpu.CompilerParams(
            collective_id=0, vmem_limit_bytes=63 * 1024 * 1024
        ),
    )(x, Wq, Wo, k_t, v_t)
